# Initial kernel scaffold; baseline (speedup 1.0000x reference)
import jax
import jax.numpy as jnp
from jax.experimental import pallas as pl


def kernel(tokens, table):
    raise NotImplementedError("write your pallas kernel here")


# speedup vs baseline: 2.8702x; 2.8702x over previous
import functools

import jax
import jax.numpy as jnp
from jax import lax
from jax.experimental import pallas as pl
from jax.experimental.pallas import tpu as pltpu
from jax.experimental.pallas import tpu_sc as plsc

_BATCH = 4096
_SEQ = 200
_DT = 64
_DE = 32
_DOUT = _DT - 1 + _DE
_IDXC = 63
_B = _BATCH * _SEQ
_NC = 2
_NS = 16
_NW = _NC * _NS
_BPW = _B // _NW
_CH = 256
_NCH = _BPW // _CH


def _sc_body(tokens_hbm, table_hbm, out_hbm, tok_v, idx_v, g_v, out_v, sem):
    wid = lax.axis_index("s") * _NC + lax.axis_index("c")
    base0 = wid * _BPW

    def chunk(i, carry):
        base = base0 + i * _CH
        pltpu.sync_copy(tokens_hbm.at[pl.ds(base, _CH), :], tok_v)

        def ext(j, c):
            rows = lax.iota(jnp.int32, 16) + j * 16
            cols = jnp.full((16,), _IDXC, jnp.int32)
            v = plsc.load_gather(tok_v, [rows, cols])
            idx_v[pl.ds(j * 16, 16)] = v.astype(jnp.int32)
            return c

        lax.fori_loop(0, _CH // 16, ext, 0)

        pltpu.async_copy(table_hbm.at[idx_v], g_v, sem).wait()

        def asm(r, c):
            out_v[r, pl.ds(0, 16)] = tok_v[r, pl.ds(0, 16)]
            out_v[r, pl.ds(16, 16)] = tok_v[r, pl.ds(16, 16)]
            out_v[r, pl.ds(32, 16)] = tok_v[r, pl.ds(32, 16)]
            out_v[r, pl.ds(48, 16)] = tok_v[r, pl.ds(48, 16)]
            out_v[r, pl.ds(63, 16)] = g_v[r, pl.ds(0, 16)]
            out_v[r, pl.ds(79, 16)] = g_v[r, pl.ds(16, 16)]
            return c

        lax.fori_loop(0, _CH, asm, 0)

        pltpu.sync_copy(out_v, out_hbm.at[pl.ds(base, _CH), :])
        return carry

    lax.fori_loop(0, _NCH, chunk, 0)


_sc_call = functools.partial(
    pl.kernel,
    out_type=jax.ShapeDtypeStruct((_B, _DOUT), jnp.float32),
    mesh=plsc.VectorSubcoreMesh(core_axis_name="c", subcore_axis_name="s"),
    compiler_params=pltpu.CompilerParams(needs_layout_passes=False),
    scratch_types=[
        pltpu.VMEM((_CH, _DT), jnp.float32),
        pltpu.VMEM((_CH,), jnp.int32),
        pltpu.VMEM((_CH, 128), jnp.float32),
        pltpu.VMEM((_CH, _DOUT), jnp.float32),
        pltpu.SemaphoreType.DMA,
    ],
)(_sc_body)


def kernel(tokens, table):
    tflat = tokens.reshape(_B, _DT)
    table128 = jnp.pad(table, ((0, 0), (0, 128 - _DE)))
    out = _sc_call(tflat, table128)
    return out.reshape(_BATCH, _SEQ, _DOUT)
